# Initial kernel scaffold; baseline (speedup 1.0000x reference)
import jax
import jax.numpy as jnp
from jax.experimental import pallas as pl


def kernel(h, edge_weight, W1, b1, g1, be1, W2, b2, g2, be2, g3, be3, edge_index):
    raise NotImplementedError("write your pallas kernel here")


# speedup vs baseline: 3.3264x; 3.3264x over previous
import functools

import jax
import jax.numpy as jnp
from jax import lax
from jax.experimental import pallas as pl
from jax.experimental.pallas import tpu as pltpu
from jax.experimental.pallas import tpu_sc as plsc

N = 10000
E = 320000
D = 128
L = 2

NC = 2
NS = 16
NW = NC * NS
EPT = E // NW
K = 80
NCH = EPT // K
NP = 10240
RPT = NP // NS


def _sc_agg_body(h_hbm, src_hbm, dst_hbm, w_hbm, zero_hbm, out_hbm,
                 src_v, dst_v, w_v, rows_v, acc_sh, sem):
    cid = lax.axis_index("c")
    sid = lax.axis_index("s")
    tid = cid * NS + sid
    pltpu.sync_copy(zero_hbm, acc_sh.at[pl.ds(sid * RPT, RPT)])
    plsc.subcore_barrier()
    base = tid * EPT

    def chunk(c, carry):
        off = base + c * K
        pltpu.sync_copy(src_hbm.at[pl.ds(off, K)], src_v)
        pltpu.sync_copy(dst_hbm.at[pl.ds(off, K)], dst_v)
        pltpu.sync_copy(w_hbm.at[pl.ds(off, K)], w_v)
        pltpu.async_copy(h_hbm.at[src_v], rows_v, sem).wait()

        def scale(i, carry2):
            wb = plsc.load_gather(w_v, [jnp.full((16,), i, jnp.int32)])
            for j in range(D // 16):
                sl = pl.ds(j * 16, 16)
                rows_v[i, sl] = rows_v[i, sl] * wb
            return carry2

        lax.fori_loop(0, K, scale, 0)
        pltpu.sync_copy(rows_v, acc_sh.at[dst_v], add=True)
        return carry

    lax.fori_loop(0, NCH, chunk, 0)
    plsc.subcore_barrier()
    row0 = cid * NP + sid * RPT
    pltpu.sync_copy(acc_sh.at[pl.ds(sid * RPT, RPT)],
                    out_hbm.at[pl.ds(row0, RPT)])


@jax.jit
def _sc_agg(h, src, dst, w, zero_rows):
    mesh = plsc.VectorSubcoreMesh(core_axis_name="c", subcore_axis_name="s")
    return pl.kernel(
        _sc_agg_body,
        out_type=jax.ShapeDtypeStruct((NC * NP, D), jnp.float32),
        mesh=mesh,
        scratch_types=[
            pltpu.VMEM((K,), jnp.int32),
            pltpu.VMEM((K,), jnp.int32),
            pltpu.VMEM((K,), jnp.float32),
            pltpu.VMEM((K, D), jnp.float32),
            pltpu.VMEM_SHARED((NP, D), jnp.float32),
            pltpu.SemaphoreType.DMA,
        ],
        compiler_params=pltpu.CompilerParams(use_tc_tiling_on_sc=False,
                                             needs_layout_passes=False),
    )(h, src, dst, w, zero_rows)


def _bn(x, g, b):
    m = jnp.mean(x, axis=0, keepdims=True)
    v = jnp.var(x, axis=0, keepdims=True)
    return (x - m) / jnp.sqrt(v + 1e-5) * g + b


def _dense_body(agg_ref, h_ref, W1_ref, b1_ref, g1_ref, be1_ref,
                W2_ref, b2_ref, g2_ref, be2_ref, g3_ref, be3_ref, out_ref):
    x = agg_ref[0] + agg_ref[1] + h_ref[...]
    x = jnp.dot(x, W1_ref[...].T, preferred_element_type=jnp.float32)
    x = x + b1_ref[...]
    x = jax.nn.relu(_bn(x, g1_ref[...], be1_ref[...]))
    x = jnp.dot(x, W2_ref[...].T, preferred_element_type=jnp.float32)
    x = x + b2_ref[...]
    x = jax.nn.relu(_bn(x, g2_ref[...], be2_ref[...]))
    out_ref[...] = jax.nn.relu(_bn(x, g3_ref[...], be3_ref[...]))


@jax.jit
def _dense(agg2, h, W1l, b1l, g1l, be1l, W2l, b2l, g2l, be2l, g3l, be3l):
    return pl.pallas_call(
        _dense_body,
        out_shape=jax.ShapeDtypeStruct((N, D), jnp.float32),
    )(agg2, h, W1l, b1l, g1l, be1l, W2l, b2l, g2l, be2l, g3l, be3l)


def kernel(h, edge_weight, W1, b1, g1, be1, W2, b2, g2, be2, g3, be3, edge_index):
    src = edge_index[0]
    dst = edge_index[1]
    zero_rows = jnp.zeros((RPT, D), jnp.float32)
    for l in range(L):
        agg = _sc_agg(h, src, dst, edge_weight[l], zero_rows)
        agg2 = agg.reshape(NC, NP, D)[:, :N]
        h = _dense(agg2, h,
                   W1[l], b1[l].reshape(1, D), g1[l].reshape(1, D),
                   be1[l].reshape(1, D), W2[l], b2[l].reshape(1, D),
                   g2[l].reshape(1, D), be2[l].reshape(1, D),
                   g3[l].reshape(1, D), be3[l].reshape(1, D))
    return h
